# Initial kernel scaffold; baseline (speedup 1.0000x reference)
import jax
import jax.numpy as jnp
from jax.experimental import pallas as pl


def kernel(metadata, waveform_features, edge_index, batch, W_meta, b_meta, W_wave, b_wave, W_comb, b_comb, gat_W0, gat_W1, gat_W2, att_src0, att_src1, att_src2, att_dst0, att_dst1, att_dst2, gat_b0, gat_b1, gat_b2, bn_g0, bn_g1, bn_g2, bn_b0, bn_b1, bn_b2, W_lat1, b_lat1, W_lat2, b_lat2, W_lon1, b_lon1, W_lon2, b_lon2):
    raise NotImplementedError("write your pallas kernel here")


# speedup vs baseline: 41.4975x; 41.4975x over previous
import functools

import jax
import jax.numpy as jnp
from jax import lax
from jax.experimental import pallas as pl
from jax.experimental.pallas import tpu as pltpu
from jax.experimental.pallas import tpu_sc as plsc

N = 10000
E = 320000
HID = 64
HEADS = 2
G = 64
F = HEADS * HID

NC, NS, L = 2, 16, 16
NW = NC * NS
FPT = F // NW
EPT = E // NW
KB = 6400
NCHUNK = E // KB
NPAIR = NCHUNK // 2

_SC_MESH = plsc.VectorSubcoreMesh(
    core_axis_name="c", subcore_axis_name="s", num_cores=NC, num_subcores=NS)
_SC_PARAMS = pltpu.CompilerParams(needs_layout_passes=False)


def _leaky(x):
    return jnp.maximum(x, 0.2 * x)


def _dot(a, b):
    return jnp.dot(a, b, preferred_element_type=jnp.float32,
                   precision=jax.lax.Precision.HIGHEST)


@functools.partial(
    pl.kernel, mesh=_SC_MESH, compiler_params=_SC_PARAMS,
    out_type=(
        jax.ShapeDtypeStruct((2 * E,), jnp.float32),
        jax.ShapeDtypeStruct((2, NW, N), jnp.float32),
    ),
    scratch_types=[
        pltpu.VMEM((N,), jnp.float32), pltpu.VMEM((N,), jnp.float32),
        pltpu.VMEM((N,), jnp.float32), pltpu.VMEM((N,), jnp.float32),
        pltpu.VMEM((N,), jnp.float32), pltpu.VMEM((N,), jnp.float32),
        pltpu.VMEM((N,), jnp.float32), pltpu.VMEM((N,), jnp.float32),
        pltpu.VMEM((EPT,), jnp.int32), pltpu.VMEM((EPT,), jnp.int32),
        pltpu.VMEM((EPT,), jnp.float32), pltpu.VMEM((EPT,), jnp.float32),
    ],
)
def _sc_edge_weights(asT_hbm, adT_hbm, cT_hbm, src_hbm, dst_hbm,
                     ee01_hbm, den_hbm,
                     as0_v, as1_v, ad0_v, ad1_v, c0_v, c1_v,
                     den0_v, den1_v, src_v, dst_v, ee0_v, ee1_v):
    wid = lax.axis_index("s") * NC + lax.axis_index("c")
    base = wid * EPT
    pltpu.sync_copy(asT_hbm.at[0], as0_v)
    pltpu.sync_copy(asT_hbm.at[1], as1_v)
    pltpu.sync_copy(adT_hbm.at[0], ad0_v)
    pltpu.sync_copy(adT_hbm.at[1], ad1_v)
    pltpu.sync_copy(cT_hbm.at[0], c0_v)
    pltpu.sync_copy(cT_hbm.at[1], c1_v)
    pltpu.sync_copy(src_hbm.at[pl.ds(base, EPT)], src_v)
    pltpu.sync_copy(dst_hbm.at[pl.ds(base, EPT)], dst_v)

    def zero(i, _):
        z = jnp.zeros((L,), jnp.float32)
        den0_v[pl.ds(i * L, L)] = z
        den1_v[pl.ds(i * L, L)] = z
        return 0
    lax.fori_loop(0, N // L, zero, 0, unroll=4)

    def body(j, _):
        s16 = src_v[pl.ds(j * L, L)]
        d16 = dst_v[pl.ds(j * L, L)]
        e0 = _leaky(plsc.load_gather(as0_v, [s16]) +
                    plsc.load_gather(ad0_v, [d16]))
        ee0 = jnp.exp(e0 - plsc.load_gather(c0_v, [d16]))
        ee0_v[pl.ds(j * L, L)] = ee0
        plsc.addupdate_scatter(den0_v, [d16], ee0)
        e1 = _leaky(plsc.load_gather(as1_v, [s16]) +
                    plsc.load_gather(ad1_v, [d16]))
        ee1 = jnp.exp(e1 - plsc.load_gather(c1_v, [d16]))
        ee1_v[pl.ds(j * L, L)] = ee1
        plsc.addupdate_scatter(den1_v, [d16], ee1)
        return 0
    lax.fori_loop(0, EPT // L, body, 0)

    pltpu.sync_copy(ee0_v, ee01_hbm.at[pl.ds(base, EPT)])
    pltpu.sync_copy(ee1_v, ee01_hbm.at[pl.ds(E + base, EPT)])
    pltpu.sync_copy(den0_v, den_hbm.at[0, wid])
    pltpu.sync_copy(den1_v, den_hbm.at[1, wid])


@functools.partial(
    pl.kernel, mesh=_SC_MESH, compiler_params=_SC_PARAMS,
    out_type=jax.ShapeDtypeStruct((F, N), jnp.float32),
    scratch_types=[
        pltpu.VMEM((N,), jnp.float32), pltpu.VMEM((N,), jnp.float32),
        pltpu.VMEM((N,), jnp.float32), pltpu.VMEM((N,), jnp.float32),
        pltpu.VMEM((N,), jnp.float32), pltpu.VMEM((N,), jnp.float32),
        pltpu.VMEM((N,), jnp.float32), pltpu.VMEM((N,), jnp.float32),
        pltpu.VMEM((KB,), jnp.int32), pltpu.VMEM((KB,), jnp.int32),
        pltpu.VMEM((KB,), jnp.float32),
        pltpu.VMEM((KB,), jnp.int32), pltpu.VMEM((KB,), jnp.int32),
        pltpu.VMEM((KB,), jnp.float32),
        pltpu.SemaphoreType.DMA((6,)),
    ],
)
def _sc_messages(hT_hbm, src_hbm, dst_hbm, ee01_hbm, num_hbm,
                 h0_v, h1_v, h2_v, h3_v, n0_v, n1_v, n2_v, n3_v,
                 srcA_v, dstA_v, eeA_v, srcB_v, dstB_v, eeB_v, sems):
    wid = lax.axis_index("s") * NC + lax.axis_index("c")
    f0 = wid * FPT
    head = wid // (NW // HEADS)
    ee_base = head * E

    def start(c, src_v, dst_v, ee_v, s0):
        off = c * KB
        pltpu.async_copy(src_hbm.at[pl.ds(off, KB)], src_v, sems.at[s0])
        pltpu.async_copy(dst_hbm.at[pl.ds(off, KB)], dst_v, sems.at[s0 + 1])
        pltpu.async_copy(ee01_hbm.at[pl.ds(ee_base + off, KB)], ee_v,
                         sems.at[s0 + 2])

    def wait(src_v, dst_v, ee_v, s0):
        pltpu.make_async_copy(src_hbm.at[pl.ds(0, KB)], src_v, sems.at[s0]).wait()
        pltpu.make_async_copy(dst_hbm.at[pl.ds(0, KB)], dst_v, sems.at[s0 + 1]).wait()
        pltpu.make_async_copy(ee01_hbm.at[pl.ds(0, KB)], ee_v, sems.at[s0 + 2]).wait()

    start(0, srcA_v, dstA_v, eeA_v, 0)

    pltpu.sync_copy(hT_hbm.at[f0], h0_v)
    pltpu.sync_copy(hT_hbm.at[f0 + 1], h1_v)
    pltpu.sync_copy(hT_hbm.at[f0 + 2], h2_v)
    pltpu.sync_copy(hT_hbm.at[f0 + 3], h3_v)

    def zero(i, _):
        z = jnp.zeros((L,), jnp.float32)
        n0_v[pl.ds(i * L, L)] = z
        n1_v[pl.ds(i * L, L)] = z
        n2_v[pl.ds(i * L, L)] = z
        n3_v[pl.ds(i * L, L)] = z
        return 0
    lax.fori_loop(0, N // L, zero, 0, unroll=4)

    def process(src_v, dst_v, ee_v):
        def step(j, _):
            s16 = src_v[pl.ds(j * L, L)]
            d16 = dst_v[pl.ds(j * L, L)]
            e16 = ee_v[pl.ds(j * L, L)]
            plsc.addupdate_scatter(n0_v, [d16], plsc.load_gather(h0_v, [s16]) * e16)
            plsc.addupdate_scatter(n1_v, [d16], plsc.load_gather(h1_v, [s16]) * e16)
            plsc.addupdate_scatter(n2_v, [d16], plsc.load_gather(h2_v, [s16]) * e16)
            plsc.addupdate_scatter(n3_v, [d16], plsc.load_gather(h3_v, [s16]) * e16)
            return 0
        lax.fori_loop(0, KB // L, step, 0)

    def pair(i, _):
        start(2 * i + 1, srcB_v, dstB_v, eeB_v, 3)
        wait(srcA_v, dstA_v, eeA_v, 0)
        process(srcA_v, dstA_v, eeA_v)

        @pl.when(i < NPAIR - 1)
        def _():
            start(2 * i + 2, srcA_v, dstA_v, eeA_v, 0)

        wait(srcB_v, dstB_v, eeB_v, 3)
        process(srcB_v, dstB_v, eeB_v)
        return 0
    lax.fori_loop(0, NPAIR, pair, 0)

    pltpu.sync_copy(n0_v, num_hbm.at[f0])
    pltpu.sync_copy(n1_v, num_hbm.at[f0 + 1])
    pltpu.sync_copy(n2_v, num_hbm.at[f0 + 2])
    pltpu.sync_copy(n3_v, num_hbm.at[f0 + 3])


def _attention_tables(hT, att_src, att_dst):
    as_rows = []
    ad_rows = []
    for h in range(HEADS):
        hh = hT[h * HID:(h + 1) * HID, :]
        as_rows.append(_dot(att_src[h:h + 1, :], hh))
        ad_rows.append(_dot(att_dst[h:h + 1, :], hh))
    a_sT = jnp.concatenate(as_rows, axis=0)
    a_dT = jnp.concatenate(ad_rows, axis=0)
    gmax = jnp.max(a_sT, axis=1, keepdims=True)
    cT = _leaky(gmax + a_dT)
    selfeeT = jnp.exp(_leaky(a_sT + a_dT) - cT)
    return a_sT, a_dT, cT, selfeeT


def _enc_pre0_body(metaT_ref, waveT_ref, WmT_ref, bm_ref, WwT_ref, bw_ref,
                   WcT_ref, bc_ref, W0T_ref, asrc_ref, adst_ref,
                   hT_ref, asT_ref, adT_ref, cT_ref, selfee_ref):
    metaT = metaT_ref[...]
    WmT = WmT_ref[...]
    m = WmT[:, 0:1] * metaT[0:1, :] + WmT[:, 1:2] * metaT[1:2, :] \
        + WmT[:, 2:3] * metaT[2:3, :] + WmT[:, 3:4] * metaT[3:4, :]
    m = jnp.maximum(m + bm_ref[...], 0.0)
    w = jnp.maximum(_dot(WwT_ref[...], waveT_ref[...]) + bw_ref[...], 0.0)
    xc = jnp.concatenate([m, w], axis=0)
    xT = jnp.maximum(_dot(WcT_ref[...], xc) + bc_ref[...], 0.0)
    hT = _dot(W0T_ref[...], xT)
    hT_ref[...] = hT
    a_sT, a_dT, cT, selfeeT = _attention_tables(hT, asrc_ref[...], adst_ref[...])
    asT_ref[...] = a_sT
    adT_ref[...] = a_dT
    cT_ref[...] = cT
    selfee_ref[...] = selfeeT


def _enc_pre0(metaT, waveT, W_meta, b_meta, W_wave, b_wave, W_comb, b_comb,
              gat_W0, att_src0, att_dst0):
    return pl.pallas_call(
        _enc_pre0_body,
        out_shape=(
            jax.ShapeDtypeStruct((F, N), jnp.float32),
            jax.ShapeDtypeStruct((2, N), jnp.float32),
            jax.ShapeDtypeStruct((2, N), jnp.float32),
            jax.ShapeDtypeStruct((2, N), jnp.float32),
            jax.ShapeDtypeStruct((2, N), jnp.float32),
        ),
    )(metaT, waveT, W_meta.T, b_meta.reshape(-1, 1), W_wave.T,
      b_wave.reshape(-1, 1), W_comb.T, b_comb.reshape(-1, 1), gat_W0.T,
      att_src0, att_dst0)


def _gat_post(numT, denP, selfeeT, hT, bias, g, bb, xresT):
    den0 = jnp.sum(denP[0], axis=0, keepdims=True) + selfeeT[0:1, :] + 1e-16
    den1 = jnp.sum(denP[1], axis=0, keepdims=True) + selfeeT[1:2, :] + 1e-16
    out0 = (numT[0:HID, :] + selfeeT[0:1, :] * hT[0:HID, :]) / den0
    out1 = (numT[HID:, :] + selfeeT[1:2, :] * hT[HID:, :]) / den1
    out = jnp.concatenate([out0, out1], axis=0) + bias
    mu = jnp.mean(out, axis=1, keepdims=True)
    xc = out - mu
    var = jnp.mean(xc * xc, axis=1, keepdims=True)
    y = xc / jnp.sqrt(var + 1e-5) * g + bb
    if xresT is not None:
        y = y + xresT
    return jnp.maximum(y, 0.0)


def _make_post_pre_body(has_res):
    def body(numT_ref, denP_ref, selfee_ref, hT_ref, b_ref, g_ref,
             bb_ref, WT_ref, asrc_ref, adst_ref, *rest):
        if has_res:
            xres_ref, xT_ref, hTn_ref, asT_ref, adT_ref, cT_ref, selfeen_ref = rest
            xresT = xres_ref[...]
        else:
            xT_ref, hTn_ref, asT_ref, adT_ref, cT_ref, selfeen_ref = rest
            xresT = None
        x = _gat_post(numT_ref[...], denP_ref[...], selfee_ref[...], hT_ref[...],
                      b_ref[...], g_ref[...], bb_ref[...], xresT)
        xT_ref[...] = x
        hTn = _dot(WT_ref[...], x)
        hTn_ref[...] = hTn
        a_sT, a_dT, cT, selfeeT = _attention_tables(hTn, asrc_ref[...], adst_ref[...])
        asT_ref[...] = a_sT
        adT_ref[...] = a_dT
        cT_ref[...] = cT
        selfeen_ref[...] = selfeeT
    return body


def _post_pre(numT, denP, selfeeT, hT, bias, g, bb, W_next, att_src, att_dst,
              xresT):
    args = [numT, denP, selfeeT, hT, bias.reshape(-1, 1), g.reshape(-1, 1),
            bb.reshape(-1, 1), W_next.T, att_src, att_dst]
    if xresT is not None:
        args.append(xresT)
    return pl.pallas_call(
        _make_post_pre_body(xresT is not None),
        out_shape=(
            jax.ShapeDtypeStruct((F, N), jnp.float32),
            jax.ShapeDtypeStruct((F, N), jnp.float32),
            jax.ShapeDtypeStruct((2, N), jnp.float32),
            jax.ShapeDtypeStruct((2, N), jnp.float32),
            jax.ShapeDtypeStruct((2, N), jnp.float32),
            jax.ShapeDtypeStruct((2, N), jnp.float32),
        ),
    )(*args)


def _post_pool_body(numT_ref, denP_ref, selfee_ref, hT_ref, b_ref, g_ref,
                    bb_ref, xres_ref, batch_ref,
                    Wlat1T_ref, blat1_ref, Wlat2T_ref, blat2_ref,
                    Wlon1T_ref, blon1_ref, Wlon2T_ref, blon2_ref,
                    latT_ref, lonT_ref):
    x = _gat_post(numT_ref[...], denP_ref[...], selfee_ref[...], hT_ref[...],
                  b_ref[...], g_ref[...], bb_ref[...], xres_ref[...])
    batch = batch_ref[...]
    gids = jax.lax.broadcasted_iota(jnp.int32, (N, G), 1)
    onehot = (batch == gids).astype(jnp.float32)
    sums = _dot(x, onehot)
    counts = jnp.sum(onehot, axis=0, keepdims=True)
    xg = sums / jnp.maximum(counts, 1.0)
    t1 = jnp.maximum(_dot(Wlat1T_ref[...], xg) + blat1_ref[...], 0.0)
    latT_ref[...] = _dot(Wlat2T_ref[...], t1) + blat2_ref[...]
    t2 = jnp.maximum(_dot(Wlon1T_ref[...], xg) + blon1_ref[...], 0.0)
    lonT_ref[...] = _dot(Wlon2T_ref[...], t2) + blon2_ref[...]


def _post_pool(numT, denP, selfeeT, hT, bias, g, bb, xresT, batch2d,
               W_lat1, b_lat1, W_lat2, b_lat2, W_lon1, b_lon1, W_lon2, b_lon2):
    return pl.pallas_call(
        _post_pool_body,
        out_shape=(
            jax.ShapeDtypeStruct((1, G), jnp.float32),
            jax.ShapeDtypeStruct((1, G), jnp.float32),
        ),
    )(numT, denP, selfeeT, hT, bias.reshape(-1, 1), g.reshape(-1, 1),
      bb.reshape(-1, 1), xresT, batch2d,
      W_lat1.T, b_lat1.reshape(-1, 1), W_lat2.T, b_lat2.reshape(-1, 1),
      W_lon1.T, b_lon1.reshape(-1, 1), W_lon2.T, b_lon2.reshape(-1, 1))


def kernel(metadata, waveform_features, edge_index, batch, W_meta, b_meta, W_wave, b_wave, W_comb, b_comb, gat_W0, gat_W1, gat_W2, att_src0, att_src1, att_src2, att_dst0, att_dst1, att_dst2, gat_b0, gat_b1, gat_b2, bn_g0, bn_g1, bn_g2, bn_b0, bn_b1, bn_b2, W_lat1, b_lat1, W_lat2, b_lat2, W_lon1, b_lon1, W_lon2, b_lon2):
    metaT = metadata.T
    waveT = waveform_features.T
    srcs = edge_index[0]
    dsts = edge_index[1]
    batch2d = batch.reshape(N, 1)

    hT, asT, adT, cT, selfeeT = _enc_pre0(
        metaT, waveT, W_meta, b_meta, W_wave, b_wave, W_comb, b_comb,
        gat_W0, att_src0, att_dst0)

    gBs = [gat_b0, gat_b1, gat_b2]
    gGs = [bn_g0, bn_g1, bn_g2]
    gbs = [bn_b0, bn_b1, bn_b2]
    Ws = [None, gat_W1, gat_W2]
    aSs = [None, att_src1, att_src2]
    aDs = [None, att_dst1, att_dst2]

    xresT = None
    for i in range(3):
        ee01, denP = _sc_edge_weights(asT, adT, cT, srcs, dsts)
        numT = _sc_messages(hT, srcs, dsts, ee01)
        if i < 2:
            xT, hT2, asT, adT, cT, selfee2 = _post_pre(
                numT, denP, selfeeT, hT, gBs[i], gGs[i], gbs[i],
                Ws[i + 1], aSs[i + 1], aDs[i + 1], xresT)
            xresT = xT
            hT = hT2
            selfeeT = selfee2
        else:
            latT, lonT = _post_pool(
                numT, denP, selfeeT, hT, gBs[i], gGs[i], gbs[i], xresT,
                batch2d, W_lat1, b_lat1, W_lat2, b_lat2,
                W_lon1, b_lon1, W_lon2, b_lon2)
    return (latT.reshape(G, 1), lonT.reshape(G, 1))


# speedup vs baseline: 103.3289x; 2.4900x over previous
import functools

import jax
import jax.numpy as jnp
from jax import lax
from jax.experimental import pallas as pl
from jax.experimental.pallas import tpu as pltpu
from jax.experimental.pallas import tpu_sc as plsc

N = 10000
E = 320000
HID = 64
HEADS = 2
G = 64
F = HEADS * HID

NC, NS, L = 2, 16, 16
NW = NC * NS
FPT = F // NW
EPT = E // NW
KB = 6400
NCHUNK = E // KB
NPAIR = NCHUNK // 2

_SC_MESH = plsc.VectorSubcoreMesh(
    core_axis_name="c", subcore_axis_name="s", num_cores=NC, num_subcores=NS)
_SC_PARAMS = pltpu.CompilerParams(needs_layout_passes=False)


def _leaky(x):
    return jnp.maximum(x, 0.2 * x)


def _dot(a, b):
    return jnp.dot(a, b, preferred_element_type=jnp.float32,
                   precision=jax.lax.Precision.HIGHEST)


@functools.partial(
    pl.kernel, mesh=_SC_MESH, compiler_params=_SC_PARAMS,
    out_type=(
        jax.ShapeDtypeStruct((2 * E,), jnp.float32),
        jax.ShapeDtypeStruct((2, NW, N), jnp.float32),
    ),
    scratch_types=[
        pltpu.VMEM((N,), jnp.float32), pltpu.VMEM((N,), jnp.float32),
        pltpu.VMEM((N,), jnp.float32), pltpu.VMEM((N,), jnp.float32),
        pltpu.VMEM((N,), jnp.float32), pltpu.VMEM((N,), jnp.float32),
        pltpu.VMEM((N,), jnp.float32), pltpu.VMEM((N,), jnp.float32),
        pltpu.VMEM((EPT,), jnp.int32), pltpu.VMEM((EPT,), jnp.int32),
        pltpu.VMEM((EPT,), jnp.float32), pltpu.VMEM((EPT,), jnp.float32),
    ],
)
def _sc_edge_weights(asT_hbm, adT_hbm, cT_hbm, src_hbm, dst_hbm,
                     ee01_hbm, den_hbm,
                     as0_v, as1_v, ad0_v, ad1_v, c0_v, c1_v,
                     den0_v, den1_v, src_v, dst_v, ee0_v, ee1_v):
    wid = lax.axis_index("s") * NC + lax.axis_index("c")
    base = wid * EPT
    pltpu.sync_copy(asT_hbm.at[0], as0_v)
    pltpu.sync_copy(asT_hbm.at[1], as1_v)
    pltpu.sync_copy(adT_hbm.at[0], ad0_v)
    pltpu.sync_copy(adT_hbm.at[1], ad1_v)
    pltpu.sync_copy(cT_hbm.at[0], c0_v)
    pltpu.sync_copy(cT_hbm.at[1], c1_v)
    pltpu.sync_copy(src_hbm.at[pl.ds(base, EPT)], src_v)
    pltpu.sync_copy(dst_hbm.at[pl.ds(base, EPT)], dst_v)

    def zero(i, _):
        z = jnp.zeros((L,), jnp.float32)
        den0_v[pl.ds(i * L, L)] = z
        den1_v[pl.ds(i * L, L)] = z
        return 0
    lax.fori_loop(0, N // L, zero, 0, unroll=4)

    @plsc.parallel_loop(0, EPT // L, 1, unroll=4)
    def body(j):
        s16 = src_v[pl.ds(j * L, L)]
        d16 = dst_v[pl.ds(j * L, L)]
        e0 = _leaky(plsc.load_gather(as0_v, [s16]) +
                    plsc.load_gather(ad0_v, [d16]))
        ee0 = jnp.exp(e0 - plsc.load_gather(c0_v, [d16]))
        ee0_v[pl.ds(j * L, L)] = ee0
        plsc.addupdate_scatter(den0_v, [d16], ee0)
        e1 = _leaky(plsc.load_gather(as1_v, [s16]) +
                    plsc.load_gather(ad1_v, [d16]))
        ee1 = jnp.exp(e1 - plsc.load_gather(c1_v, [d16]))
        ee1_v[pl.ds(j * L, L)] = ee1
        plsc.addupdate_scatter(den1_v, [d16], ee1)

    pltpu.sync_copy(ee0_v, ee01_hbm.at[pl.ds(base, EPT)])
    pltpu.sync_copy(ee1_v, ee01_hbm.at[pl.ds(E + base, EPT)])
    pltpu.sync_copy(den0_v, den_hbm.at[0, wid])
    pltpu.sync_copy(den1_v, den_hbm.at[1, wid])


@functools.partial(
    pl.kernel, mesh=_SC_MESH, compiler_params=_SC_PARAMS,
    out_type=jax.ShapeDtypeStruct((F, N), jnp.float32),
    scratch_types=[
        pltpu.VMEM((N,), jnp.float32), pltpu.VMEM((N,), jnp.float32),
        pltpu.VMEM((N,), jnp.float32), pltpu.VMEM((N,), jnp.float32),
        pltpu.VMEM((N,), jnp.float32), pltpu.VMEM((N,), jnp.float32),
        pltpu.VMEM((N,), jnp.float32), pltpu.VMEM((N,), jnp.float32),
        pltpu.VMEM((KB,), jnp.int32), pltpu.VMEM((KB,), jnp.int32),
        pltpu.VMEM((KB,), jnp.float32),
        pltpu.VMEM((KB,), jnp.int32), pltpu.VMEM((KB,), jnp.int32),
        pltpu.VMEM((KB,), jnp.float32),
        pltpu.SemaphoreType.DMA((6,)),
    ],
)
def _sc_messages(hT_hbm, src_hbm, dst_hbm, ee01_hbm, num_hbm,
                 h0_v, h1_v, h2_v, h3_v, n0_v, n1_v, n2_v, n3_v,
                 srcA_v, dstA_v, eeA_v, srcB_v, dstB_v, eeB_v, sems):
    wid = lax.axis_index("s") * NC + lax.axis_index("c")
    f0 = wid * FPT
    head = wid // (NW // HEADS)
    ee_base = head * E

    def start(c, src_v, dst_v, ee_v, s0):
        off = c * KB
        pltpu.async_copy(src_hbm.at[pl.ds(off, KB)], src_v, sems.at[s0])
        pltpu.async_copy(dst_hbm.at[pl.ds(off, KB)], dst_v, sems.at[s0 + 1])
        pltpu.async_copy(ee01_hbm.at[pl.ds(ee_base + off, KB)], ee_v,
                         sems.at[s0 + 2])

    def wait(src_v, dst_v, ee_v, s0):
        pltpu.make_async_copy(src_hbm.at[pl.ds(0, KB)], src_v, sems.at[s0]).wait()
        pltpu.make_async_copy(dst_hbm.at[pl.ds(0, KB)], dst_v, sems.at[s0 + 1]).wait()
        pltpu.make_async_copy(ee01_hbm.at[pl.ds(0, KB)], ee_v, sems.at[s0 + 2]).wait()

    start(0, srcA_v, dstA_v, eeA_v, 0)

    pltpu.sync_copy(hT_hbm.at[f0], h0_v)
    pltpu.sync_copy(hT_hbm.at[f0 + 1], h1_v)
    pltpu.sync_copy(hT_hbm.at[f0 + 2], h2_v)
    pltpu.sync_copy(hT_hbm.at[f0 + 3], h3_v)

    def zero(i, _):
        z = jnp.zeros((L,), jnp.float32)
        n0_v[pl.ds(i * L, L)] = z
        n1_v[pl.ds(i * L, L)] = z
        n2_v[pl.ds(i * L, L)] = z
        n3_v[pl.ds(i * L, L)] = z
        return 0
    lax.fori_loop(0, N // L, zero, 0, unroll=4)

    def process(src_v, dst_v, ee_v):
        @plsc.parallel_loop(0, KB // L, 1, unroll=4)
        def step(j):
            s16 = src_v[pl.ds(j * L, L)]
            d16 = dst_v[pl.ds(j * L, L)]
            e16 = ee_v[pl.ds(j * L, L)]
            plsc.addupdate_scatter(n0_v, [d16], plsc.load_gather(h0_v, [s16]) * e16)
            plsc.addupdate_scatter(n1_v, [d16], plsc.load_gather(h1_v, [s16]) * e16)
            plsc.addupdate_scatter(n2_v, [d16], plsc.load_gather(h2_v, [s16]) * e16)
            plsc.addupdate_scatter(n3_v, [d16], plsc.load_gather(h3_v, [s16]) * e16)

    def pair(i, _):
        start(2 * i + 1, srcB_v, dstB_v, eeB_v, 3)
        wait(srcA_v, dstA_v, eeA_v, 0)
        process(srcA_v, dstA_v, eeA_v)

        @pl.when(i < NPAIR - 1)
        def _():
            start(2 * i + 2, srcA_v, dstA_v, eeA_v, 0)

        wait(srcB_v, dstB_v, eeB_v, 3)
        process(srcB_v, dstB_v, eeB_v)
        return 0
    lax.fori_loop(0, NPAIR, pair, 0)

    pltpu.sync_copy(n0_v, num_hbm.at[f0])
    pltpu.sync_copy(n1_v, num_hbm.at[f0 + 1])
    pltpu.sync_copy(n2_v, num_hbm.at[f0 + 2])
    pltpu.sync_copy(n3_v, num_hbm.at[f0 + 3])


def _attention_tables(hT, att_src, att_dst):
    as_rows = []
    ad_rows = []
    for h in range(HEADS):
        hh = hT[h * HID:(h + 1) * HID, :]
        as_rows.append(_dot(att_src[h:h + 1, :], hh))
        ad_rows.append(_dot(att_dst[h:h + 1, :], hh))
    a_sT = jnp.concatenate(as_rows, axis=0)
    a_dT = jnp.concatenate(ad_rows, axis=0)
    gmax = jnp.max(a_sT, axis=1, keepdims=True)
    cT = _leaky(gmax + a_dT)
    selfeeT = jnp.exp(_leaky(a_sT + a_dT) - cT)
    return a_sT, a_dT, cT, selfeeT


def _enc_pre0_body(metaT_ref, waveT_ref, WmT_ref, bm_ref, WwT_ref, bw_ref,
                   WcT_ref, bc_ref, W0T_ref, asrc_ref, adst_ref,
                   hT_ref, asT_ref, adT_ref, cT_ref, selfee_ref):
    metaT = metaT_ref[...]
    WmT = WmT_ref[...]
    m = WmT[:, 0:1] * metaT[0:1, :] + WmT[:, 1:2] * metaT[1:2, :] \
        + WmT[:, 2:3] * metaT[2:3, :] + WmT[:, 3:4] * metaT[3:4, :]
    m = jnp.maximum(m + bm_ref[...], 0.0)
    w = jnp.maximum(_dot(WwT_ref[...], waveT_ref[...]) + bw_ref[...], 0.0)
    xc = jnp.concatenate([m, w], axis=0)
    xT = jnp.maximum(_dot(WcT_ref[...], xc) + bc_ref[...], 0.0)
    hT = _dot(W0T_ref[...], xT)
    hT_ref[...] = hT
    a_sT, a_dT, cT, selfeeT = _attention_tables(hT, asrc_ref[...], adst_ref[...])
    asT_ref[...] = a_sT
    adT_ref[...] = a_dT
    cT_ref[...] = cT
    selfee_ref[...] = selfeeT


def _enc_pre0(metaT, waveT, W_meta, b_meta, W_wave, b_wave, W_comb, b_comb,
              gat_W0, att_src0, att_dst0):
    return pl.pallas_call(
        _enc_pre0_body,
        out_shape=(
            jax.ShapeDtypeStruct((F, N), jnp.float32),
            jax.ShapeDtypeStruct((2, N), jnp.float32),
            jax.ShapeDtypeStruct((2, N), jnp.float32),
            jax.ShapeDtypeStruct((2, N), jnp.float32),
            jax.ShapeDtypeStruct((2, N), jnp.float32),
        ),
    )(metaT, waveT, W_meta.T, b_meta.reshape(-1, 1), W_wave.T,
      b_wave.reshape(-1, 1), W_comb.T, b_comb.reshape(-1, 1), gat_W0.T,
      att_src0, att_dst0)


def _gat_post(numT, denP, selfeeT, hT, bias, g, bb, xresT):
    den0 = jnp.sum(denP[0], axis=0, keepdims=True) + selfeeT[0:1, :] + 1e-16
    den1 = jnp.sum(denP[1], axis=0, keepdims=True) + selfeeT[1:2, :] + 1e-16
    out0 = (numT[0:HID, :] + selfeeT[0:1, :] * hT[0:HID, :]) / den0
    out1 = (numT[HID:, :] + selfeeT[1:2, :] * hT[HID:, :]) / den1
    out = jnp.concatenate([out0, out1], axis=0) + bias
    mu = jnp.mean(out, axis=1, keepdims=True)
    xc = out - mu
    var = jnp.mean(xc * xc, axis=1, keepdims=True)
    y = xc / jnp.sqrt(var + 1e-5) * g + bb
    if xresT is not None:
        y = y + xresT
    return jnp.maximum(y, 0.0)


def _make_post_pre_body(has_res):
    def body(numT_ref, denP_ref, selfee_ref, hT_ref, b_ref, g_ref,
             bb_ref, WT_ref, asrc_ref, adst_ref, *rest):
        if has_res:
            xres_ref, xT_ref, hTn_ref, asT_ref, adT_ref, cT_ref, selfeen_ref = rest
            xresT = xres_ref[...]
        else:
            xT_ref, hTn_ref, asT_ref, adT_ref, cT_ref, selfeen_ref = rest
            xresT = None
        x = _gat_post(numT_ref[...], denP_ref[...], selfee_ref[...], hT_ref[...],
                      b_ref[...], g_ref[...], bb_ref[...], xresT)
        xT_ref[...] = x
        hTn = _dot(WT_ref[...], x)
        hTn_ref[...] = hTn
        a_sT, a_dT, cT, selfeeT = _attention_tables(hTn, asrc_ref[...], adst_ref[...])
        asT_ref[...] = a_sT
        adT_ref[...] = a_dT
        cT_ref[...] = cT
        selfeen_ref[...] = selfeeT
    return body


def _post_pre(numT, denP, selfeeT, hT, bias, g, bb, W_next, att_src, att_dst,
              xresT):
    args = [numT, denP, selfeeT, hT, bias.reshape(-1, 1), g.reshape(-1, 1),
            bb.reshape(-1, 1), W_next.T, att_src, att_dst]
    if xresT is not None:
        args.append(xresT)
    return pl.pallas_call(
        _make_post_pre_body(xresT is not None),
        out_shape=(
            jax.ShapeDtypeStruct((F, N), jnp.float32),
            jax.ShapeDtypeStruct((F, N), jnp.float32),
            jax.ShapeDtypeStruct((2, N), jnp.float32),
            jax.ShapeDtypeStruct((2, N), jnp.float32),
            jax.ShapeDtypeStruct((2, N), jnp.float32),
            jax.ShapeDtypeStruct((2, N), jnp.float32),
        ),
    )(*args)


def _post_pool_body(numT_ref, denP_ref, selfee_ref, hT_ref, b_ref, g_ref,
                    bb_ref, xres_ref, batch_ref,
                    Wlat1T_ref, blat1_ref, Wlat2T_ref, blat2_ref,
                    Wlon1T_ref, blon1_ref, Wlon2T_ref, blon2_ref,
                    latT_ref, lonT_ref):
    x = _gat_post(numT_ref[...], denP_ref[...], selfee_ref[...], hT_ref[...],
                  b_ref[...], g_ref[...], bb_ref[...], xres_ref[...])
    batch = batch_ref[...]
    gids = jax.lax.broadcasted_iota(jnp.int32, (N, G), 1)
    onehot = (batch == gids).astype(jnp.float32)
    sums = _dot(x, onehot)
    counts = jnp.sum(onehot, axis=0, keepdims=True)
    xg = sums / jnp.maximum(counts, 1.0)
    t1 = jnp.maximum(_dot(Wlat1T_ref[...], xg) + blat1_ref[...], 0.0)
    latT_ref[...] = _dot(Wlat2T_ref[...], t1) + blat2_ref[...]
    t2 = jnp.maximum(_dot(Wlon1T_ref[...], xg) + blon1_ref[...], 0.0)
    lonT_ref[...] = _dot(Wlon2T_ref[...], t2) + blon2_ref[...]


def _post_pool(numT, denP, selfeeT, hT, bias, g, bb, xresT, batch2d,
               W_lat1, b_lat1, W_lat2, b_lat2, W_lon1, b_lon1, W_lon2, b_lon2):
    return pl.pallas_call(
        _post_pool_body,
        out_shape=(
            jax.ShapeDtypeStruct((1, G), jnp.float32),
            jax.ShapeDtypeStruct((1, G), jnp.float32),
        ),
    )(numT, denP, selfeeT, hT, bias.reshape(-1, 1), g.reshape(-1, 1),
      bb.reshape(-1, 1), xresT, batch2d,
      W_lat1.T, b_lat1.reshape(-1, 1), W_lat2.T, b_lat2.reshape(-1, 1),
      W_lon1.T, b_lon1.reshape(-1, 1), W_lon2.T, b_lon2.reshape(-1, 1))


def kernel(metadata, waveform_features, edge_index, batch, W_meta, b_meta, W_wave, b_wave, W_comb, b_comb, gat_W0, gat_W1, gat_W2, att_src0, att_src1, att_src2, att_dst0, att_dst1, att_dst2, gat_b0, gat_b1, gat_b2, bn_g0, bn_g1, bn_g2, bn_b0, bn_b1, bn_b2, W_lat1, b_lat1, W_lat2, b_lat2, W_lon1, b_lon1, W_lon2, b_lon2):
    metaT = metadata.T
    waveT = waveform_features.T
    srcs = edge_index[0]
    dsts = edge_index[1]
    batch2d = batch.reshape(N, 1)

    hT, asT, adT, cT, selfeeT = _enc_pre0(
        metaT, waveT, W_meta, b_meta, W_wave, b_wave, W_comb, b_comb,
        gat_W0, att_src0, att_dst0)

    gBs = [gat_b0, gat_b1, gat_b2]
    gGs = [bn_g0, bn_g1, bn_g2]
    gbs = [bn_b0, bn_b1, bn_b2]
    Ws = [None, gat_W1, gat_W2]
    aSs = [None, att_src1, att_src2]
    aDs = [None, att_dst1, att_dst2]

    xresT = None
    for i in range(3):
        ee01, denP = _sc_edge_weights(asT, adT, cT, srcs, dsts)
        numT = _sc_messages(hT, srcs, dsts, ee01)
        if i < 2:
            xT, hT2, asT, adT, cT, selfee2 = _post_pre(
                numT, denP, selfeeT, hT, gBs[i], gGs[i], gbs[i],
                Ws[i + 1], aSs[i + 1], aDs[i + 1], xresT)
            xresT = xT
            hT = hT2
            selfeeT = selfee2
        else:
            latT, lonT = _post_pool(
                numT, denP, selfeeT, hT, gBs[i], gGs[i], gbs[i], xresT,
                batch2d, W_lat1, b_lat1, W_lat2, b_lat2,
                W_lon1, b_lon1, W_lon2, b_lon2)
    return (latT.reshape(G, 1), lonT.reshape(G, 1))
